# baseline (speedup 1.0000x reference)
import functools

import jax
import jax.numpy as jnp
from jax import lax
from jax.experimental import pallas as pl
from jax.experimental.pallas import tpu as pltpu
from jax.experimental.pallas import tpu_sc as plsc

_NUM_ROWS = 4096
_EMB = 64
_NUM_COLS = 3 * _EMB
_NC = 2
_NS = 16
_NW = _NC * _NS
_B = _NUM_ROWS // _NW


@functools.partial(
    pl.kernel,
    mesh=plsc.VectorSubcoreMesh(core_axis_name="c", subcore_axis_name="s"),
    out_type=jax.ShapeDtypeStruct((_NUM_ROWS, _NUM_COLS), jnp.float32),
    scratch_types=[
        pltpu.VMEM((_B,), jnp.int32),
        pltpu.VMEM((_B,), jnp.int32),
        pltpu.VMEM((_B,), jnp.int32),
        pltpu.VMEM((_B,), jnp.int32),
        pltpu.VMEM((_B,), jnp.int32),
        pltpu.VMEM((_B,), jnp.int32),
        pltpu.VMEM((_B,), jnp.int32),
        pltpu.VMEM((_B,), jnp.int32),
        pltpu.VMEM((_B, 2 * _EMB), jnp.float32),
        pltpu.VMEM((_B, 2 * _EMB), jnp.float32),
        pltpu.VMEM((_B, 2 * _EMB), jnp.float32),
        pltpu.VMEM((_B, 2 * _EMB), jnp.float32),
        pltpu.VMEM((1, 2 * _EMB), jnp.float32),
        pltpu.VMEM((_B, _NUM_COLS), jnp.float32),
        pltpu.SemaphoreType.DMA,
    ],
)
def _emb_kernel(heads_hbm, rels_hbm, tails_hbm, names_hbm, specials_hbm,
                hp_hbm, rp_hbm, tp_hbm, np_hbm,
                ho_hbm, ro_hbm, to_hbm, no_hbm, out_hbm,
                hp_v, rp_v, tp_v, np_v, ho_v, ro_v, to_v, no_v,
                h_v, r_v, t_v, n_v, spec_v, out_v, sem):
    wid = lax.axis_index("s") * _NC + lax.axis_index("c")
    base = wid * _B

    pltpu.sync_copy(hp_hbm.at[pl.ds(base, _B)], hp_v)
    pltpu.sync_copy(rp_hbm.at[pl.ds(base, _B)], rp_v)
    pltpu.sync_copy(tp_hbm.at[pl.ds(base, _B)], tp_v)
    pltpu.sync_copy(np_hbm.at[pl.ds(base, _B)], np_v)
    pltpu.sync_copy(ho_hbm.at[pl.ds(base, _B)], ho_v)
    pltpu.sync_copy(ro_hbm.at[pl.ds(base, _B)], ro_v)
    pltpu.sync_copy(to_hbm.at[pl.ds(base, _B)], to_v)
    pltpu.sync_copy(no_hbm.at[pl.ds(base, _B)], no_v)

    cps = [pltpu.async_copy(heads_hbm.at[hp_v], h_v, sem),
           pltpu.async_copy(rels_hbm.at[rp_v], r_v, sem),
           pltpu.async_copy(tails_hbm.at[tp_v], t_v, sem),
           pltpu.async_copy(names_hbm.at[np_v], n_v, sem)]
    for cp in cps:
        cp.wait()

    def grp_body(g, carry):
        r0 = g * 16
        hov = ho_v[pl.ds(r0, 16)]
        rov = ro_v[pl.ds(r0, 16)]
        tov = to_v[pl.ds(r0, 16)]
        nov = no_v[pl.ds(r0, 16)]
        for j in range(16):
            r = r0 + j
            ho, ro, to, no = hov[j], rov[j], tov[j], nov[j]
            for c in range(_EMB // 16):
                s = 16 * c
                n = n_v[r, pl.ds(no + s, 16)]
                out_v[r, pl.ds(s, 16)] = n + h_v[r, pl.ds(ho + s, 16)]
                out_v[r, pl.ds(_EMB + s, 16)] = r_v[r, pl.ds(ro + s, 16)]
                out_v[r, pl.ds(2 * _EMB + s, 16)] = (
                    n + t_v[r, pl.ds(to + s, 16)])
        return carry

    lax.fori_loop(0, _B // 16, grp_body, 0)

    @pl.when(wid == _NW - 1)
    def _fix_question_tail():
        pltpu.sync_copy(specials_hbm.at[pl.ds(0, 1)], spec_v)
        no = no_v[pl.ds(_B - 16, 16)][15]
        for c in range(_EMB // 16):
            s = 16 * c
            out_v[_B - 1, pl.ds(2 * _EMB + s, 16)] = (
                n_v[_B - 1, pl.ds(no + s, 16)]
                + spec_v[0, pl.ds(_EMB + s, 16)])

    pltpu.sync_copy(out_v, out_hbm.at[pl.ds(base, _B)])


def kernel(heads_w, relations_w, tails_w, names_w, specials_w,
           head_idx, rel_idx, tail_idx, name_idx, q_head, q_rel, q_name):
    i32 = jnp.int32
    hid = jnp.concatenate([head_idx.astype(i32), q_head.astype(i32)])
    rid = jnp.concatenate([rel_idx.astype(i32), q_rel.astype(i32)])
    tid = jnp.concatenate([tail_idx.astype(i32), jnp.zeros((1,), i32)])
    nid = jnp.concatenate([name_idx.astype(i32), q_name.astype(i32)])
    heads2 = heads_w.reshape(-1, 2 * _EMB)
    rels2 = relations_w.reshape(-1, 2 * _EMB)
    tails2 = tails_w.reshape(-1, 2 * _EMB)
    names2 = names_w.reshape(-1, 2 * _EMB)
    specials2 = specials_w.reshape(-1, 2 * _EMB)
    return _emb_kernel(
        heads2, rels2, tails2, names2, specials2,
        hid >> 1, rid >> 1, tid >> 1, nid >> 1,
        (hid & 1) << 6, (rid & 1) << 6, (tid & 1) << 6, (nid & 1) << 6)


# speedup vs baseline: 1.7408x; 1.7408x over previous
import functools

import jax
import jax.numpy as jnp
from jax import lax
from jax.experimental import pallas as pl
from jax.experimental.pallas import tpu as pltpu
from jax.experimental.pallas import tpu_sc as plsc

_NUM_ROWS = 4096
_EMB = 64
_NUM_COLS = 3 * _EMB
_NC = 2
_NS = 16
_NW = _NC * _NS
_B = _NUM_ROWS // _NW
_NG = _B // 16


@functools.partial(
    pl.kernel,
    mesh=plsc.VectorSubcoreMesh(core_axis_name="c", subcore_axis_name="s"),
    out_type=jax.ShapeDtypeStruct((_NUM_ROWS, _NUM_COLS), jnp.float32),
    scratch_types=[
        pltpu.VMEM((_B,), jnp.int32),
        pltpu.VMEM((_B,), jnp.int32),
        pltpu.VMEM((_B,), jnp.int32),
        pltpu.VMEM((_B,), jnp.int32),
        pltpu.VMEM((_B,), jnp.int32),
        pltpu.VMEM((_B,), jnp.int32),
        pltpu.VMEM((_B,), jnp.int32),
        pltpu.VMEM((_B,), jnp.int32),
        pltpu.VMEM((_B, _EMB), jnp.float32),
        pltpu.VMEM((_B, _EMB), jnp.float32),
        pltpu.VMEM((_B, _EMB), jnp.float32),
        pltpu.VMEM((_B, _EMB), jnp.float32),
        pltpu.VMEM((1, 8, _EMB), jnp.float32),
        pltpu.VMEM((_B, _NUM_COLS), jnp.float32),
        pltpu.SemaphoreType.DMA,
    ],
)
def _emb_kernel(heads_hbm, rels_hbm, tails_hbm, names_hbm, specials_hbm,
                ht_hbm, hs_hbm, tt_hbm, ts_hbm, nt_hbm, ns_hbm,
                rt_hbm, rs_hbm, out_hbm,
                ht_v, hs_v, tt_v, ts_v, nt_v, ns_v, rt_v, rs_v,
                h_v, r_v, t_v, n_v, spec_v, out_v, sem):
    wid = lax.axis_index("s") * _NC + lax.axis_index("c")
    base = wid * _B

    pltpu.sync_copy(ht_hbm.at[pl.ds(base, _B)], ht_v)
    pltpu.sync_copy(hs_hbm.at[pl.ds(base, _B)], hs_v)
    pltpu.sync_copy(tt_hbm.at[pl.ds(base, _B)], tt_v)
    pltpu.sync_copy(ts_hbm.at[pl.ds(base, _B)], ts_v)
    pltpu.sync_copy(nt_hbm.at[pl.ds(base, _B)], nt_v)
    pltpu.sync_copy(ns_hbm.at[pl.ds(base, _B)], ns_v)
    pltpu.sync_copy(rt_hbm.at[pl.ds(base, _B)], rt_v)
    pltpu.sync_copy(rs_hbm.at[pl.ds(base, _B)], rs_v)
    pltpu.sync_copy(specials_hbm, spec_v)

    def issue_body(g, carry):
        e0 = g * 16
        htv = ht_v[pl.ds(e0, 16)]
        hsv = hs_v[pl.ds(e0, 16)]
        ttv = tt_v[pl.ds(e0, 16)]
        tsv = ts_v[pl.ds(e0, 16)]
        ntv = nt_v[pl.ds(e0, 16)]
        nsv = ns_v[pl.ds(e0, 16)]
        rtv = rt_v[pl.ds(e0, 16)]
        rsv = rs_v[pl.ds(e0, 16)]
        for j in range(16):
            e = e0 + j
            pltpu.async_copy(heads_hbm.at[htv[j], hsv[j]], h_v.at[e], sem)
            pltpu.async_copy(rels_hbm.at[rtv[j], rsv[j]], r_v.at[e], sem)
            pltpu.async_copy(tails_hbm.at[ttv[j], tsv[j]], t_v.at[e], sem)
            pltpu.async_copy(names_hbm.at[ntv[j], nsv[j]], n_v.at[e], sem)
        return carry

    lax.fori_loop(0, _NG, issue_body, 0)

    def drain_body(e, carry):
        pltpu.make_async_copy(heads_hbm.at[0, 0], h_v.at[0], sem).wait()
        pltpu.make_async_copy(rels_hbm.at[0, 0], r_v.at[0], sem).wait()
        pltpu.make_async_copy(tails_hbm.at[0, 0], t_v.at[0], sem).wait()
        pltpu.make_async_copy(names_hbm.at[0, 0], n_v.at[0], sem).wait()
        return carry

    lax.fori_loop(0, _B, drain_body, 0)

    def row_body(r, carry):
        for c in range(_EMB // 16):
            s = 16 * c
            n = n_v[r, pl.ds(s, 16)]
            out_v[r, pl.ds(s, 16)] = n + h_v[r, pl.ds(s, 16)]
            out_v[r, pl.ds(_EMB + s, 16)] = r_v[r, pl.ds(s, 16)]
            out_v[r, pl.ds(2 * _EMB + s, 16)] = n + t_v[r, pl.ds(s, 16)]
        return carry

    lax.fori_loop(0, _B, row_body, 0)

    @pl.when(wid == _NW - 1)
    def _fix_question_tail():
        for c in range(_EMB // 16):
            s = 16 * c
            out_v[_B - 1, pl.ds(2 * _EMB + s, 16)] = (
                n_v[_B - 1, pl.ds(s, 16)] + spec_v[0, 1, pl.ds(s, 16)])

    pltpu.sync_copy(out_v, out_hbm.at[pl.ds(base, _B)])


def kernel(heads_w, relations_w, tails_w, names_w, specials_w,
           head_idx, rel_idx, tail_idx, name_idx, q_head, q_rel, q_name):
    i32 = jnp.int32
    hid = jnp.concatenate([head_idx.astype(i32), q_head.astype(i32)])
    rid = jnp.concatenate([rel_idx.astype(i32), q_rel.astype(i32)])
    tid = jnp.concatenate([tail_idx.astype(i32), jnp.zeros((1,), i32)])
    nid = jnp.concatenate([name_idx.astype(i32), q_name.astype(i32)])
    heads3 = heads_w.reshape(-1, 8, _EMB)
    rels3 = relations_w.reshape(-1, 8, _EMB)
    tails3 = tails_w.reshape(-1, 8, _EMB)
    names3 = names_w.reshape(-1, 8, _EMB)
    specials3 = jnp.pad(specials_w, ((0, 6), (0, 0))).reshape(1, 8, _EMB)
    return _emb_kernel(
        heads3, rels3, tails3, names3, specials3,
        hid >> 3, hid & 7, tid >> 3, tid & 7,
        nid >> 3, nid & 7, rid >> 3, rid & 7)
